# baseline (speedup 1.0000x reference)
import functools

import jax
import jax.numpy as jnp
from jax import lax
from jax.experimental import pallas as pl
from jax.experimental.pallas import tpu as pltpu
from jax.experimental.pallas import tpu_sc as plsc

H = 64
B = 16384
NC = 2
NS = 16
NW = NC * NS
BPW = B // NW
CHUNK = 128
NCHUNK = BPW // CHUNK
L = 16


def _gate_sc(x, wbt, theta):
    mesh = plsc.VectorSubcoreMesh(core_axis_name="c", subcore_axis_name="s")

    @functools.partial(
        pl.kernel,
        out_type=jax.ShapeDtypeStruct((B,), jnp.float32),
        mesh=mesh,
        scratch_types=[
            pltpu.VMEM((BPW,), jnp.int32),
            pltpu.VMEM((BPW, H), jnp.float32),
            pltpu.VMEM((H + 1, L), jnp.float32),
            pltpu.VMEM((BPW,), jnp.float32),
            pltpu.SemaphoreType.DMA,
        ],
        compiler_params=pltpu.CompilerParams(
            needs_layout_passes=False, use_tc_tiling_on_sc=False),
    )
    def k(x_hbm, wbt_hbm, theta_hbm, out_hbm, idx_v, rows_v, wbt_v, out_v,
          sem):
        wid = lax.axis_index("s") * NC + lax.axis_index("c")
        base = wid * BPW
        pltpu.sync_copy(x_hbm.at[pl.ds(base, BPW)], idx_v)
        pltpu.sync_copy(wbt_hbm, wbt_v)
        copies = [
            pltpu.async_copy(
                theta_hbm.at[idx_v.at[pl.ds(c * CHUNK, CHUNK)]],
                rows_v.at[pl.ds(c * CHUNK, CHUNK), :],
                sem,
            )
            for c in range(NCHUNK)
        ]
        for cp in copies:
            cp.wait()

        lanes = lax.iota(jnp.int32, L)
        bv = wbt_v[H, :]
        zero = jnp.zeros((L,), jnp.float32)
        def dot_body(g, carry):
            row0 = pl.multiple_of(g * L, L)
            ridx = row0 + lanes
            acc = [bv, zero, zero, zero]
            for j in range(H):
                col = plsc.load_gather(
                    rows_v, [ridx, jnp.full((L,), j, jnp.int32)])
                acc[j % 4] = acc[j % 4] + col * wbt_v[j, :]
            s = (acc[0] + acc[1]) + (acc[2] + acc[3])
            out_v[pl.ds(row0, L)] = 1.0 / (1.0 + jnp.exp(-s))
            return carry

        lax.fori_loop(0, BPW // L, dot_body, 0)
        pltpu.sync_copy(out_v, out_hbm.at[pl.ds(base, BPW)])

    return k(x, wbt, theta)


def kernel(X, Y, gate_theta, W, b):
    wbt = jnp.concatenate(
        [jnp.broadcast_to(W[:, 0][:, None], (H, L)),
         jnp.broadcast_to(b, (1, L))]).astype(jnp.float32)
    out = _gate_sc(X, wbt, gate_theta)
    return out[:, None]


# speedup vs baseline: 1.6873x; 1.6873x over previous
import functools

import jax
import jax.numpy as jnp
from jax import lax
from jax.experimental import pallas as pl
from jax.experimental.pallas import tpu as pltpu
from jax.experimental.pallas import tpu_sc as plsc

H = 64
B = 16384
NC = 2
NS = 16
NW = NC * NS
BPW = B // NW
L = 16


def _gate_sc(x, wbt, theta):
    mesh = plsc.VectorSubcoreMesh(core_axis_name="c", subcore_axis_name="s")

    @functools.partial(
        pl.kernel,
        out_type=jax.ShapeDtypeStruct((B,), jnp.float32),
        mesh=mesh,
        scratch_types=[
            pltpu.VMEM((BPW,), jnp.int32),
            pltpu.VMEM((BPW, H), jnp.float32),
            pltpu.VMEM((H + 1, L), jnp.float32),
            pltpu.VMEM((BPW,), jnp.float32),
            pltpu.SemaphoreType.DMA,
        ],
        compiler_params=pltpu.CompilerParams(needs_layout_passes=False),
    )
    def k(x_hbm, wbt_hbm, theta_hbm, out_hbm, idx_v, rows_v, wbt_v, out_v,
          sem):
        wid = lax.axis_index("s") * NC + lax.axis_index("c")
        base = wid * BPW
        pltpu.sync_copy(x_hbm.at[pl.ds(base, BPW)], idx_v)
        pltpu.sync_copy(wbt_hbm, wbt_v)

        def fire_body(g, carry):
            off = pl.multiple_of(g * L, L)
            v = idx_v[pl.ds(off, L)]
            for l in range(L):
                pltpu.async_copy(
                    theta_hbm.at[pl.ds(v[l], 1), :],
                    rows_v.at[pl.ds(off + l, 1), :],
                    sem,
                )
            return carry

        lax.fori_loop(0, BPW // L, fire_body, 0)
        pltpu.make_async_copy(
            theta_hbm.at[pl.ds(0, BPW), :], rows_v, sem).wait()

        lanes = lax.iota(jnp.int32, L)
        bv = wbt_v[H, :]
        zero = jnp.zeros((L,), jnp.float32)

        def dot_body(g, carry):
            row0 = pl.multiple_of(g * L, L)
            ridx = row0 + lanes
            acc = [bv, zero, zero, zero]
            for j in range(H):
                col = plsc.load_gather(
                    rows_v, [ridx, jnp.full((L,), j, jnp.int32)])
                acc[j % 4] = acc[j % 4] + col * wbt_v[j, :]
            s = (acc[0] + acc[1]) + (acc[2] + acc[3])
            out_v[pl.ds(row0, L)] = 1.0 / (1.0 + jnp.exp(-s))
            return carry

        lax.fori_loop(0, BPW // L, dot_body, 0)
        pltpu.sync_copy(out_v, out_hbm.at[pl.ds(base, BPW)])

    return k(x, wbt, theta)


def kernel(X, Y, gate_theta, W, b):
    wbt = jnp.concatenate(
        [jnp.broadcast_to(W[:, 0][:, None], (H, L)),
         jnp.broadcast_to(b, (1, L))]).astype(jnp.float32)
    out = _gate_sc(X, wbt, gate_theta)
    return out[:, None]


# speedup vs baseline: 1.6933x; 1.0035x over previous
import functools

import jax
import jax.numpy as jnp
from jax import lax
from jax.experimental import pallas as pl
from jax.experimental.pallas import tpu as pltpu
from jax.experimental.pallas import tpu_sc as plsc

H = 64
B = 16384
NC = 2
NS = 16
NW = NC * NS
BPW = B // NW
L = 16


def _gate_sc(x, wbt, theta):
    mesh = plsc.VectorSubcoreMesh(core_axis_name="c", subcore_axis_name="s")

    @functools.partial(
        pl.kernel,
        out_type=jax.ShapeDtypeStruct((B,), jnp.float32),
        mesh=mesh,
        scratch_types=[
            pltpu.VMEM((BPW,), jnp.int32),
            pltpu.VMEM((BPW, H), jnp.float32),
            pltpu.VMEM((H + 1, L), jnp.float32),
            pltpu.VMEM((BPW,), jnp.float32),
            pltpu.SemaphoreType.DMA,
        ],
        compiler_params=pltpu.CompilerParams(
            needs_layout_passes=False, disable_bounds_checks=True),
    )
    def k(x_hbm, wbt_hbm, theta_hbm, out_hbm, idx_v, rows_v, wbt_v, out_v,
          sem):
        wid = lax.axis_index("s") * NC + lax.axis_index("c")
        base = wid * BPW
        pltpu.sync_copy(x_hbm.at[pl.ds(base, BPW)], idx_v)
        pltpu.sync_copy(wbt_hbm, wbt_v)

        def fire_body(g, carry):
            off = pl.multiple_of(g * L, L)
            v = idx_v[pl.ds(off, L)]
            for l in range(L):
                pltpu.async_copy(
                    theta_hbm.at[pl.ds(v[l], 1), :],
                    rows_v.at[pl.ds(off + l, 1), :],
                    sem,
                )
            return carry

        lax.fori_loop(0, BPW // L, fire_body, 0)
        pltpu.make_async_copy(
            theta_hbm.at[pl.ds(0, BPW), :], rows_v, sem).wait()

        lanes = lax.iota(jnp.int32, L)
        bv = wbt_v[H, :]
        zero = jnp.zeros((L,), jnp.float32)

        def dot_body(g, carry):
            row0 = pl.multiple_of(g * L, L)
            ridx = row0 + lanes
            acc = [bv, zero, zero, zero]
            for j in range(H):
                col = plsc.load_gather(
                    rows_v, [ridx, jnp.full((L,), j, jnp.int32)])
                acc[j % 4] = acc[j % 4] + col * wbt_v[j, :]
            s = (acc[0] + acc[1]) + (acc[2] + acc[3])
            out_v[pl.ds(row0, L)] = 1.0 / (1.0 + jnp.exp(-s))
            return carry

        lax.fori_loop(0, BPW // L, dot_body, 0)
        pltpu.sync_copy(out_v, out_hbm.at[pl.ds(base, BPW)])

    return k(x, wbt, theta)


def kernel(X, Y, gate_theta, W, b):
    wbt = jnp.concatenate(
        [jnp.broadcast_to(W[:, 0][:, None], (H, L)),
         jnp.broadcast_to(b, (1, L))]).astype(jnp.float32)
    out = _gate_sc(X, wbt, gate_theta)
    return out[:, None]
